# baseline (speedup 1.0000x reference)
import math

import jax
import jax.numpy as jnp
from jax import lax
from jax.experimental import pallas as pl
from jax.experimental.pallas import tpu as pltpu
from jax.experimental.pallas import tpu_sc as plsc

_B, _S, _D = 8, 4096, 1024
_K = 64
_H = 256
_SF = 128
_SH = 256

_T1 = 2048
_NS1 = _S // _T1
_T2 = 8192
_NT2 = (_K * _D) // _T2

_SQRT2_INV = 0.7071067811865476
_LOG1E4 = math.log(10000.0)

_NC = 2
_NSUB = 16
_NW = _NC * _NSUB
_L = 16
_COLS_PER_W = (_K * _D) // _NW
_CW = 128
_NCH = _COLS_PER_W // _CW
_GV = 4


def _pool_copy_body(seq_ref, emb_ref, out_ref, acc_ref):
    b = pl.program_id(0)
    s = pl.program_id(1)
    x = emb_ref[0]
    out_ref[0] = x
    seqlen = seq_ref[b]
    rows = s * _T1 + jax.lax.broadcasted_iota(jnp.int32, (_T1, 1), 0)
    mask = (rows < seqlen).astype(jnp.float32)
    partial = jnp.sum(x * mask, axis=0)[None, :]

    @pl.when(s == 0)
    def _():
        acc_ref[0] = partial

    @pl.when(s != 0)
    def _():
        acc_ref[0] = acc_ref[0] + partial


def _mlp_sigma_body(t_ref, Ws1_ref, bs1_ref, hs_ref):
    half = _SF // 2
    k_iota = jax.lax.broadcasted_iota(jnp.int32, (1, half), 1).astype(jnp.float32)
    freqs = jnp.exp(-(_LOG1E4 / half) * k_iota)
    angles = t_ref[...] * freqs
    sigma = jnp.concatenate([jnp.cos(angles), jnp.sin(angles)], axis=1)
    zs = jnp.dot(sigma, Ws1_ref[...],
                 preferred_element_type=jnp.float32,
                 precision=jax.lax.Precision.HIGHEST) + bs1_ref[...]
    hs_ref[...] = zs * jax.nn.sigmoid(zs)


_GATHER_DNUMS = lax.GatherDimensionNumbers(
    offset_dims=(), collapsed_slice_dims=(0,), start_index_map=(0,))


def _lane_splat(vec, j):
    idx = jnp.full((_L, 1), j, jnp.int32)
    return lax.gather(vec, idx, _GATHER_DNUMS, slice_sizes=(1,),
                      mode=lax.GatherScatterMode.PROMISE_IN_BOUNDS)


def _sc_sigma_body(hs_hbm, ws2_hbm, out_hbm, hs_v, wbuf0, wbuf1, out_v,
                   sem0, sem1):
    wid = lax.axis_index("s") * _NC + lax.axis_index("c")
    base = wid * _COLS_PER_W
    pltpu.sync_copy(hs_hbm, hs_v)
    wbufs = (wbuf0, wbuf1)
    sems = (sem0, sem1)

    def _start(c, buf, sem):
        pltpu.async_copy(ws2_hbm.at[:, pl.ds(base + c * _CW, _CW)], buf, sem)

    def _wait(buf, sem):
        pltpu.make_async_copy(ws2_hbm.at[:, pl.ds(base, _CW)], buf, sem).wait()

    def _compute(c, buf):
        for g in range(_CW // (_GV * _L)):
            def kbody(kb, accs):
                accs = list(accs)
                for b in range(_B):
                    hs16 = hs_v[b, pl.ds(kb * _L, _L)]
                    for j in range(_L):
                        hsbj = _lane_splat(hs16, j)
                        k = kb * _L + j
                        for v in range(_GV):
                            w = buf[k, pl.ds(g * _GV * _L + v * _L, _L)]
                            accs[b * _GV + v] = accs[b * _GV + v] + hsbj * w
                return tuple(accs)

            accs = lax.fori_loop(
                0, _H // _L, kbody,
                tuple(jnp.zeros((_L,), jnp.float32) for _ in range(_B * _GV)))
            for b in range(_B):
                for v in range(_GV):
                    out_v[b, pl.ds(c * _CW + g * _GV * _L + v * _L, _L)] = (
                        accs[b * _GV + v])

    _start(0, wbufs[0], sems[0])

    def pair_body(cp, _):
        c0 = cp * 2
        _start(c0 + 1, wbufs[1], sems[1])
        _wait(wbufs[0], sems[0])
        _compute(c0, wbufs[0])

        @pl.when(cp < (_NCH // 2) - 1)
        def _():
            _start(c0 + 2, wbufs[0], sems[0])

        _wait(wbufs[1], sems[1])
        _compute(c0 + 1, wbufs[1])
        return 0

    lax.fori_loop(0, _NCH // 2, pair_body, 0)
    pltpu.sync_copy(out_v, out_hbm.at[:, pl.ds(base, _COLS_PER_W)])


def _mlp_cond_body(pooled_ref, seqf_ref, W1_ref, b1_ref, h_ref):
    denom = jnp.maximum(seqf_ref[...], 1.0)
    pooled = pooled_ref[:, 0, :] / denom
    z = jnp.dot(pooled, W1_ref[...],
                preferred_element_type=jnp.float32,
                precision=jax.lax.Precision.HIGHEST) + b1_ref[...]
    h_ref[...] = 0.5 * z * (1.0 + jax.lax.erf(z * _SQRT2_INV))


def _postfix_body(h_ref, W2_ref, b2_ref, bs2_ref, sc_ref, pf_ref):
    pf = jnp.dot(h_ref[...], W2_ref[...],
                 preferred_element_type=jnp.float32,
                 precision=jax.lax.Precision.HIGHEST)
    pf_ref[...] = pf + sc_ref[...] + b2_ref[...] + bs2_ref[...]


def _splice_body(seq_ref, src_ref, pf_ref, out_ref):
    b = pl.program_id(0)
    j = pl.program_id(1)
    seqlen = seq_ref[b]
    r = jax.lax.rem(seqlen, _K)
    pf = pf_ref[0]
    rolled = pltpu.roll(pf, r, 0)
    rows = jax.lax.broadcasted_iota(jnp.int32, (_K, 1), 0)
    is_first = (j == 0)
    keep_new = ((rows >= r) & is_first) | ((rows < r) & jnp.logical_not(is_first))
    out_ref[0] = jnp.where(keep_new, rolled, src_ref[0])


def kernel(crossattn_emb, crossattn_seqlens, timesteps, W1, b1, W2, b2,
           Ws1, bs1, Ws2, bs2):
    seq_i32 = crossattn_seqlens.astype(jnp.int32)
    t2d = timesteps.astype(jnp.float32).reshape(_B, 1)

    hs = pl.pallas_call(
        _mlp_sigma_body,
        out_shape=jax.ShapeDtypeStruct((_B, _SH), jnp.float32),
    )(t2d, Ws1, bs1.reshape(1, _SH))

    mesh = plsc.VectorSubcoreMesh(core_axis_name="c", subcore_axis_name="s")
    pf_sc = pl.kernel(
        _sc_sigma_body,
        out_type=jax.ShapeDtypeStruct((_B, _K * _D), jnp.float32),
        mesh=mesh,
        scratch_types=[
            pltpu.VMEM((_B, _SH), jnp.float32),
            pltpu.VMEM((_H, _CW), jnp.float32),
            pltpu.VMEM((_H, _CW), jnp.float32),
            pltpu.VMEM((_B, _COLS_PER_W), jnp.float32),
            pltpu.SemaphoreType.DMA,
            pltpu.SemaphoreType.DMA,
        ],
        compiler_params=pltpu.CompilerParams(needs_layout_passes=False),
    )(hs, Ws2)

    grid1 = pltpu.PrefetchScalarGridSpec(
        num_scalar_prefetch=1,
        grid=(_B, _NS1),
        in_specs=[pl.BlockSpec((1, _T1, _D), lambda b, s, seq: (b, s, 0))],
        out_specs=[
            pl.BlockSpec((1, _T1, _D), lambda b, s, seq: (b, s, 0)),
            pl.BlockSpec((1, 1, _D), lambda b, s, seq: (b, 0, 0)),
        ],
    )
    out1, pooled_sum = pl.pallas_call(
        _pool_copy_body,
        grid_spec=grid1,
        out_shape=[
            jax.ShapeDtypeStruct((_B, _S, _D), jnp.float32),
            jax.ShapeDtypeStruct((_B, 1, _D), jnp.float32),
        ],
        compiler_params=pltpu.CompilerParams(
            dimension_semantics=("arbitrary", "arbitrary")),
    )(seq_i32, crossattn_emb)

    seqf = seq_i32.astype(jnp.float32).reshape(_B, 1)
    h = pl.pallas_call(
        _mlp_cond_body,
        out_shape=jax.ShapeDtypeStruct((_B, _H), jnp.float32),
    )(pooled_sum, seqf, W1, b1.reshape(1, _H))

    pf = pl.pallas_call(
        _postfix_body,
        grid=(_NT2,),
        in_specs=[
            pl.BlockSpec((_B, _H), lambda t: (0, 0)),
            pl.BlockSpec((_H, _T2), lambda t: (0, t)),
            pl.BlockSpec((1, _T2), lambda t: (0, t)),
            pl.BlockSpec((1, _T2), lambda t: (0, t)),
            pl.BlockSpec((_B, _T2), lambda t: (0, t)),
        ],
        out_specs=pl.BlockSpec((_B, _T2), lambda t: (0, t)),
        out_shape=jax.ShapeDtypeStruct((_B, _K * _D), jnp.float32),
        compiler_params=pltpu.CompilerParams(
            dimension_semantics=("arbitrary",)),
    )(h, W2, b2.reshape(1, _K * _D), bs2.reshape(1, _K * _D), pf_sc)
    pf3 = pf.reshape(_B, _K, _D)

    grid4 = pltpu.PrefetchScalarGridSpec(
        num_scalar_prefetch=1,
        grid=(_B, 2),
        in_specs=[
            pl.BlockSpec((1, _K, _D), lambda b, j, seq: (b, seq[b] // _K + j, 0)),
            pl.BlockSpec((1, _K, _D), lambda b, j, seq: (b, 0, 0)),
        ],
        out_specs=pl.BlockSpec((1, _K, _D), lambda b, j, seq: (b, seq[b] // _K + j, 0)),
    )
    out = pl.pallas_call(
        _splice_body,
        grid_spec=grid4,
        out_shape=jax.ShapeDtypeStruct((_B, _S, _D), jnp.float32),
        input_output_aliases={1: 0},
        compiler_params=pltpu.CompilerParams(
            dimension_semantics=("arbitrary", "arbitrary")),
    )(seq_i32, out1, pf3)
    return out


# speedup vs baseline: 1.2846x; 1.2846x over previous
import math

import jax
import jax.numpy as jnp
from jax import lax
from jax.experimental import pallas as pl
from jax.experimental.pallas import tpu as pltpu
from jax.experimental.pallas import tpu_sc as plsc

_B, _S, _D = 8, 4096, 1024
_K = 64
_H = 256
_SF = 128
_SH = 256

_T1 = 2048
_NS1 = _S // _T1
_T2 = 8192
_NT2 = (_K * _D) // _T2

_SQRT2_INV = 0.7071067811865476
_LOG1E4 = math.log(10000.0)

_NC = 2
_NSUB = 16
_NW = _NC * _NSUB
_L = 16
_SC_COLS = (_K * _D) // 2
_SC_NT = _SC_COLS // _T2
_COLS_PER_W = _SC_COLS // _NW
_CW = 128
_NCH = _COLS_PER_W // _CW
_GV = 4


def _pool_copy_body(seq_ref, emb_ref, out_ref, acc_ref):
    b = pl.program_id(0)
    s = pl.program_id(1)
    x = emb_ref[0]
    out_ref[0] = x
    seqlen = seq_ref[b]
    rows = s * _T1 + jax.lax.broadcasted_iota(jnp.int32, (_T1, 1), 0)
    mask = (rows < seqlen).astype(jnp.float32)
    partial = jnp.sum(x * mask, axis=0)[None, :]

    @pl.when(s == 0)
    def _():
        acc_ref[0] = partial

    @pl.when(s != 0)
    def _():
        acc_ref[0] = acc_ref[0] + partial


def _mlp_sigma_body(t_ref, Ws1_ref, bs1_ref, hs_ref):
    half = _SF // 2
    k_iota = jax.lax.broadcasted_iota(jnp.int32, (1, half), 1).astype(jnp.float32)
    freqs = jnp.exp(-(_LOG1E4 / half) * k_iota)
    angles = t_ref[...] * freqs
    sigma = jnp.concatenate([jnp.cos(angles), jnp.sin(angles)], axis=1)
    zs = jnp.dot(sigma, Ws1_ref[...],
                 preferred_element_type=jnp.float32,
                 precision=jax.lax.Precision.HIGHEST) + bs1_ref[...]
    hs_ref[...] = zs * jax.nn.sigmoid(zs)


_GATHER_DNUMS = lax.GatherDimensionNumbers(
    offset_dims=(), collapsed_slice_dims=(0,), start_index_map=(0,))


def _lane_splat(vec, j):
    idx = jnp.full((_L, 1), j, jnp.int32)
    return lax.gather(vec, idx, _GATHER_DNUMS, slice_sizes=(1,),
                      mode=lax.GatherScatterMode.PROMISE_IN_BOUNDS)


def _sc_sigma_body(hs_hbm, ws2_hbm, out_hbm, hs_v, wbuf0, wbuf1, out_v,
                   sem0, sem1):
    wid = lax.axis_index("s") * _NC + lax.axis_index("c")
    base = wid * _COLS_PER_W
    pltpu.sync_copy(hs_hbm, hs_v)
    wbufs = (wbuf0, wbuf1)
    sems = (sem0, sem1)

    def _start(c, buf, sem):
        pltpu.async_copy(ws2_hbm.at[:, pl.ds(base + c * _CW, _CW)], buf, sem)

    def _wait(buf, sem):
        pltpu.make_async_copy(ws2_hbm.at[:, pl.ds(base, _CW)], buf, sem).wait()

    def _compute(c, buf):
        for g in range(_CW // (_GV * _L)):
            def kbody(kb, accs):
                accs = list(accs)
                for b in range(_B):
                    hs16 = hs_v[b, pl.ds(kb * _L, _L)]
                    for j in range(_L):
                        hsbj = _lane_splat(hs16, j)
                        k = kb * _L + j
                        for v in range(_GV):
                            w = buf[k, pl.ds(g * _GV * _L + v * _L, _L)]
                            accs[b * _GV + v] = accs[b * _GV + v] + hsbj * w
                return tuple(accs)

            accs = lax.fori_loop(
                0, _H // _L, kbody,
                tuple(jnp.zeros((_L,), jnp.float32) for _ in range(_B * _GV)))
            for b in range(_B):
                for v in range(_GV):
                    out_v[b, pl.ds(c * _CW + g * _GV * _L + v * _L, _L)] = (
                        accs[b * _GV + v])

    _start(0, wbufs[0], sems[0])

    def pair_body(cp, _):
        c0 = cp * 2
        _start(c0 + 1, wbufs[1], sems[1])
        _wait(wbufs[0], sems[0])
        _compute(c0, wbufs[0])

        @pl.when(cp < (_NCH // 2) - 1)
        def _():
            _start(c0 + 2, wbufs[0], sems[0])

        _wait(wbufs[1], sems[1])
        _compute(c0 + 1, wbufs[1])
        return 0

    lax.fori_loop(0, _NCH // 2, pair_body, 0)
    pltpu.sync_copy(out_v, out_hbm.at[:, pl.ds(base, _COLS_PER_W)])


def _mlp_cond_body(pooled_ref, seqf_ref, W1_ref, b1_ref, h_ref):
    denom = jnp.maximum(seqf_ref[...], 1.0)
    pooled = pooled_ref[:, 0, :] / denom
    z = jnp.dot(pooled, W1_ref[...],
                preferred_element_type=jnp.float32,
                precision=jax.lax.Precision.HIGHEST) + b1_ref[...]
    h_ref[...] = 0.5 * z * (1.0 + jax.lax.erf(z * _SQRT2_INV))


def _postfix_body(h_ref, hs_ref, W2_ref, Ws2_ref, b2_ref, bs2_ref, sc_ref,
                  pf_ref):
    t = pl.program_id(0)
    pf = jnp.dot(h_ref[...], W2_ref[...],
                 preferred_element_type=jnp.float32,
                 precision=jax.lax.Precision.HIGHEST)
    sig_tc = jnp.dot(hs_ref[...], Ws2_ref[...],
                     preferred_element_type=jnp.float32,
                     precision=jax.lax.Precision.HIGHEST)
    sig = jnp.where(t < _SC_NT, sc_ref[...], sig_tc)
    pf_ref[...] = pf + sig + b2_ref[...] + bs2_ref[...]


def _splice_body(seq_ref, src_ref, pf_ref, out_ref):
    b = pl.program_id(0)
    j = pl.program_id(1)
    seqlen = seq_ref[b]
    r = jax.lax.rem(seqlen, _K)
    pf = pf_ref[0]
    rolled = pltpu.roll(pf, r, 0)
    rows = jax.lax.broadcasted_iota(jnp.int32, (_K, 1), 0)
    is_first = (j == 0)
    keep_new = ((rows >= r) & is_first) | ((rows < r) & jnp.logical_not(is_first))
    out_ref[0] = jnp.where(keep_new, rolled, src_ref[0])


def kernel(crossattn_emb, crossattn_seqlens, timesteps, W1, b1, W2, b2,
           Ws1, bs1, Ws2, bs2):
    seq_i32 = crossattn_seqlens.astype(jnp.int32)
    t2d = timesteps.astype(jnp.float32).reshape(_B, 1)

    hs = pl.pallas_call(
        _mlp_sigma_body,
        out_shape=jax.ShapeDtypeStruct((_B, _SH), jnp.float32),
    )(t2d, Ws1, bs1.reshape(1, _SH))

    mesh = plsc.VectorSubcoreMesh(core_axis_name="c", subcore_axis_name="s")
    pf_sc = pl.kernel(
        _sc_sigma_body,
        out_type=jax.ShapeDtypeStruct((_B, _SC_COLS), jnp.float32),
        mesh=mesh,
        scratch_types=[
            pltpu.VMEM((_B, _SH), jnp.float32),
            pltpu.VMEM((_H, _CW), jnp.float32),
            pltpu.VMEM((_H, _CW), jnp.float32),
            pltpu.VMEM((_B, _COLS_PER_W), jnp.float32),
            pltpu.SemaphoreType.DMA,
            pltpu.SemaphoreType.DMA,
        ],
        compiler_params=pltpu.CompilerParams(needs_layout_passes=False),
    )(hs, Ws2)

    grid1 = pltpu.PrefetchScalarGridSpec(
        num_scalar_prefetch=1,
        grid=(_B, _NS1),
        in_specs=[pl.BlockSpec((1, _T1, _D), lambda b, s, seq: (b, s, 0))],
        out_specs=[
            pl.BlockSpec((1, _T1, _D), lambda b, s, seq: (b, s, 0)),
            pl.BlockSpec((1, 1, _D), lambda b, s, seq: (b, 0, 0)),
        ],
    )
    out1, pooled_sum = pl.pallas_call(
        _pool_copy_body,
        grid_spec=grid1,
        out_shape=[
            jax.ShapeDtypeStruct((_B, _S, _D), jnp.float32),
            jax.ShapeDtypeStruct((_B, 1, _D), jnp.float32),
        ],
        compiler_params=pltpu.CompilerParams(
            dimension_semantics=("arbitrary", "arbitrary")),
    )(seq_i32, crossattn_emb)

    seqf = seq_i32.astype(jnp.float32).reshape(_B, 1)
    h = pl.pallas_call(
        _mlp_cond_body,
        out_shape=jax.ShapeDtypeStruct((_B, _H), jnp.float32),
    )(pooled_sum, seqf, W1, b1.reshape(1, _H))

    pf = pl.pallas_call(
        _postfix_body,
        grid=(_NT2,),
        in_specs=[
            pl.BlockSpec((_B, _H), lambda t: (0, 0)),
            pl.BlockSpec((_B, _SH), lambda t: (0, 0)),
            pl.BlockSpec((_H, _T2), lambda t: (0, t)),
            pl.BlockSpec((_H, _T2), lambda t: (0, jnp.maximum(t, _SC_NT))),
            pl.BlockSpec((1, _T2), lambda t: (0, t)),
            pl.BlockSpec((1, _T2), lambda t: (0, t)),
            pl.BlockSpec((_B, _T2), lambda t: (0, jnp.minimum(t, _SC_NT - 1))),
        ],
        out_specs=pl.BlockSpec((_B, _T2), lambda t: (0, t)),
        out_shape=jax.ShapeDtypeStruct((_B, _K * _D), jnp.float32),
        compiler_params=pltpu.CompilerParams(
            dimension_semantics=("arbitrary",)),
    )(h, hs, W2, Ws2, b2.reshape(1, _K * _D), bs2.reshape(1, _K * _D), pf_sc)
    pf3 = pf.reshape(_B, _K, _D)

    grid4 = pltpu.PrefetchScalarGridSpec(
        num_scalar_prefetch=1,
        grid=(_B, 2),
        in_specs=[
            pl.BlockSpec((1, _K, _D), lambda b, j, seq: (b, seq[b] // _K + j, 0)),
            pl.BlockSpec((1, _K, _D), lambda b, j, seq: (b, 0, 0)),
        ],
        out_specs=pl.BlockSpec((1, _K, _D), lambda b, j, seq: (b, seq[b] // _K + j, 0)),
    )
    out = pl.pallas_call(
        _splice_body,
        grid_spec=grid4,
        out_shape=jax.ShapeDtypeStruct((_B, _S, _D), jnp.float32),
        input_output_aliases={1: 0},
        compiler_params=pltpu.CompilerParams(
            dimension_semantics=("arbitrary", "arbitrary")),
    )(seq_i32, out1, pf3)
    return out


# speedup vs baseline: 1.5634x; 1.2170x over previous
import math

import jax
import jax.numpy as jnp
from jax.experimental import pallas as pl
from jax.experimental.pallas import tpu as pltpu

_B, _S, _D = 8, 4096, 1024
_K = 64
_H = 256
_SF = 128
_SH = 256

_T1 = 2048
_NS1 = _S // _T1
_T2 = 8192
_NT2 = (_K * _D) // _T2

_SQRT2_INV = 0.7071067811865476
_LOG1E4 = math.log(10000.0)


def _pool_copy_body(seq_ref, emb_ref, out_ref, acc_ref):
    b = pl.program_id(0)
    s = pl.program_id(1)
    x = emb_ref[0]
    out_ref[0] = x
    seqlen = seq_ref[b]
    rows = s * _T1 + jax.lax.broadcasted_iota(jnp.int32, (_T1, 1), 0)
    mask = (rows < seqlen).astype(jnp.float32)
    partial = jnp.sum(x * mask, axis=0)[None, :]

    @pl.when(s == 0)
    def _():
        acc_ref[0] = partial

    @pl.when(s != 0)
    def _():
        acc_ref[0] = acc_ref[0] + partial


def _postfix_body(pooled_ref, seqf_ref, t_ref, W1_ref, b1_ref, Ws1_ref,
                  bs1_ref, W2_ref, b2_ref, Ws2_ref, bs2_ref, pf_ref,
                  h_scr, hs_scr):
    t = pl.program_id(0)

    @pl.when(t == 0)
    def _():
        denom = jnp.maximum(seqf_ref[...], 1.0)
        pooled = pooled_ref[:, 0, :] / denom
        z = jnp.dot(pooled, W1_ref[...],
                    preferred_element_type=jnp.float32,
                    precision=jax.lax.Precision.HIGHEST) + b1_ref[...]
        h_scr[...] = 0.5 * z * (1.0 + jax.lax.erf(z * _SQRT2_INV))
        half = _SF // 2
        k_iota = jax.lax.broadcasted_iota(
            jnp.int32, (1, half), 1).astype(jnp.float32)
        freqs = jnp.exp(-(_LOG1E4 / half) * k_iota)
        angles = t_ref[...] * freqs
        sigma = jnp.concatenate([jnp.cos(angles), jnp.sin(angles)], axis=1)
        zs = jnp.dot(sigma, Ws1_ref[...],
                     preferred_element_type=jnp.float32,
                     precision=jax.lax.Precision.HIGHEST) + bs1_ref[...]
        hs_scr[...] = zs * jax.nn.sigmoid(zs)

    pf = jnp.dot(h_scr[...], W2_ref[...],
                 preferred_element_type=jnp.float32,
                 precision=jax.lax.Precision.HIGHEST)
    pf = pf + jnp.dot(hs_scr[...], Ws2_ref[...],
                      preferred_element_type=jnp.float32,
                      precision=jax.lax.Precision.HIGHEST)
    pf_ref[...] = pf + b2_ref[...] + bs2_ref[...]


def _splice_body(seq_ref, src_ref, pf_ref, out_ref):
    b = pl.program_id(0)
    j = pl.program_id(1)
    seqlen = seq_ref[b]
    r = jax.lax.rem(seqlen, _K)
    pf = pf_ref[0]
    rolled = pltpu.roll(pf, r, 0)
    rows = jax.lax.broadcasted_iota(jnp.int32, (_K, 1), 0)
    is_first = (j == 0)
    keep_new = ((rows >= r) & is_first) | ((rows < r) & jnp.logical_not(is_first))
    out_ref[0] = jnp.where(keep_new, rolled, src_ref[0])


def kernel(crossattn_emb, crossattn_seqlens, timesteps, W1, b1, W2, b2,
           Ws1, bs1, Ws2, bs2):
    seq_i32 = crossattn_seqlens.astype(jnp.int32)

    grid1 = pltpu.PrefetchScalarGridSpec(
        num_scalar_prefetch=1,
        grid=(_B, _NS1),
        in_specs=[pl.BlockSpec((1, _T1, _D), lambda b, s, seq: (b, s, 0))],
        out_specs=[
            pl.BlockSpec((1, _T1, _D), lambda b, s, seq: (b, s, 0)),
            pl.BlockSpec((1, 1, _D), lambda b, s, seq: (b, 0, 0)),
        ],
    )
    out1, pooled_sum = pl.pallas_call(
        _pool_copy_body,
        grid_spec=grid1,
        out_shape=[
            jax.ShapeDtypeStruct((_B, _S, _D), jnp.float32),
            jax.ShapeDtypeStruct((_B, 1, _D), jnp.float32),
        ],
        compiler_params=pltpu.CompilerParams(
            dimension_semantics=("arbitrary", "arbitrary")),
    )(seq_i32, crossattn_emb)

    seqf = seq_i32.astype(jnp.float32).reshape(_B, 1)
    t2d = timesteps.astype(jnp.float32).reshape(_B, 1)
    pf = pl.pallas_call(
        _postfix_body,
        grid=(_NT2,),
        in_specs=[
            pl.BlockSpec((_B, 1, _D), lambda t: (0, 0, 0)),
            pl.BlockSpec((_B, 1), lambda t: (0, 0)),
            pl.BlockSpec((_B, 1), lambda t: (0, 0)),
            pl.BlockSpec((_D, _H), lambda t: (0, 0)),
            pl.BlockSpec((1, _H), lambda t: (0, 0)),
            pl.BlockSpec((_SF, _SH), lambda t: (0, 0)),
            pl.BlockSpec((1, _SH), lambda t: (0, 0)),
            pl.BlockSpec((_H, _T2), lambda t: (0, t)),
            pl.BlockSpec((1, _T2), lambda t: (0, t)),
            pl.BlockSpec((_SH, _T2), lambda t: (0, t)),
            pl.BlockSpec((1, _T2), lambda t: (0, t)),
        ],
        out_specs=pl.BlockSpec((_B, _T2), lambda t: (0, t)),
        out_shape=jax.ShapeDtypeStruct((_B, _K * _D), jnp.float32),
        scratch_shapes=[
            pltpu.VMEM((_B, _H), jnp.float32),
            pltpu.VMEM((_B, _SH), jnp.float32),
        ],
        compiler_params=pltpu.CompilerParams(
            dimension_semantics=("arbitrary",)),
    )(pooled_sum, seqf, t2d, W1, b1.reshape(1, _H), Ws1, bs1.reshape(1, _SH),
      W2, b2.reshape(1, _K * _D), Ws2, bs2.reshape(1, _K * _D))
    pf3 = pf.reshape(_B, _K, _D)

    grid4 = pltpu.PrefetchScalarGridSpec(
        num_scalar_prefetch=1,
        grid=(_B, 2),
        in_specs=[
            pl.BlockSpec((1, _K, _D), lambda b, j, seq: (b, seq[b] // _K + j, 0)),
            pl.BlockSpec((1, _K, _D), lambda b, j, seq: (b, 0, 0)),
        ],
        out_specs=pl.BlockSpec((1, _K, _D), lambda b, j, seq: (b, seq[b] // _K + j, 0)),
    )
    out = pl.pallas_call(
        _splice_body,
        grid_spec=grid4,
        out_shape=jax.ShapeDtypeStruct((_B, _S, _D), jnp.float32),
        input_output_aliases={1: 0},
        compiler_params=pltpu.CompilerParams(
            dimension_semantics=("arbitrary", "arbitrary")),
    )(seq_i32, out1, pf3)
    return out
